# baseline (speedup 1.0000x reference)
import functools

import jax
import jax.numpy as jnp
from jax import lax
from jax.experimental import pallas as pl
from jax.experimental.pallas import tpu as pltpu
from jax.experimental.pallas import tpu_sc as plsc

_CH = 64


def _make_sc_kernel(B, S, F, NW, SEG, R):
    mesh = plsc.VectorSubcoreMesh(core_axis_name="c", subcore_axis_name="s")

    @functools.partial(
        pl.kernel,
        out_type=jax.ShapeDtypeStruct((B, S, F), jnp.float32),
        mesh=mesh,
        scratch_types=[
            pltpu.VMEM((_CH, F), jnp.float32),
            pltpu.VMEM((8, F), jnp.float32),
            pltpu.VMEM((1, 16), jnp.int32),
        ],
    )
    def sc_kernel(x_hbm, cuts_hbm, fill_hbm, out_hbm, fillbuf, tilebuf, cutbuf):
        wid = lax.axis_index("c") * 16 + lax.axis_index("s")
        b = wid // SEG
        r0 = (wid % SEG) * R
        pltpu.sync_copy(fill_hbm, fillbuf)
        pltpu.sync_copy(cuts_hbm.at[wid], cutbuf)
        cut = cutbuf[0, :][0]

        def chunk(i, carry):
            row = r0 + i * _CH
            live = jnp.clip(cut - i * _CH, 0, _CH)

            @pl.when(live == _CH)
            def _():
                pltpu.sync_copy(
                    x_hbm.at[b, pl.ds(row, _CH)], out_hbm.at[b, pl.ds(row, _CH)]
                )

            @pl.when(live == 0)
            def _():
                pltpu.sync_copy(fillbuf, out_hbm.at[b, pl.ds(row, _CH)])

            @pl.when(jnp.logical_and(live > 0, live < _CH))
            def _():
                pltpu.sync_copy(
                    x_hbm.at[b, pl.ds(row, _CH)], out_hbm.at[b, pl.ds(row, _CH)]
                )
                a = (live // 8) * 8
                frac = live - a

                @pl.when(frac > 0)
                def _():
                    pltpu.sync_copy(x_hbm.at[b, pl.ds(row + a, 8)], tilebuf)

                    def patch(r, c):
                        for j in range(F // 16):
                            tilebuf[r, pl.ds(j * 16, 16)] = fillbuf[
                                0, pl.ds(j * 16, 16)
                            ]
                        return c

                    lax.fori_loop(frac, 8, patch, 0)
                    pltpu.sync_copy(tilebuf, out_hbm.at[b, pl.ds(row + a, 8)])

                fill0 = a + jnp.where(frac > 0, 8, 0)
                rem = _CH - fill0
                off = fill0
                for sz in (32, 16, 8):
                    take = rem >= sz
                    cur = off

                    @pl.when(take)
                    def _(sz=sz, cur=cur):
                        pltpu.sync_copy(
                            fillbuf.at[pl.ds(0, sz)],
                            out_hbm.at[b, pl.ds(row + cur, sz)],
                        )

                    off = jnp.where(take, off + sz, off)
                    rem = jnp.where(take, rem - sz, rem)

            return carry

        lax.fori_loop(0, R // _CH, chunk, 0)

    return sc_kernel


def kernel(x, lens, output_mask):
    B, S, F = x.shape
    NW = 32
    SEG = NW // B
    R = S // SEG
    lens_i = lens.astype(jnp.int32)
    wids = jnp.arange(NW, dtype=jnp.int32)
    cut = jnp.clip(lens_i[wids // SEG] - (wids % SEG) * R, 0, R)
    cuts3 = jnp.broadcast_to(cut[:, None, None], (NW, 1, 16))
    fill = jnp.broadcast_to(output_mask[None, :], (_CH, F))
    return _make_sc_kernel(B, S, F, NW, SEG, R)(x, cuts3, fill)


# speedup vs baseline: 5.7696x; 5.7696x over previous
import functools

import jax
import jax.numpy as jnp
from jax import lax
from jax.experimental import pallas as pl
from jax.experimental.pallas import tpu as pltpu
from jax.experimental.pallas import tpu_sc as plsc

_CH = 32


def _make_sc_kernel(B, S, F, NW, SEG, R):
    mesh = plsc.VectorSubcoreMesh(core_axis_name="c", subcore_axis_name="s")
    n_chunks = R // _CH

    @functools.partial(
        pl.kernel,
        out_type=jax.ShapeDtypeStruct((B, S, F), jnp.float32),
        mesh=mesh,
        scratch_types=[
            pltpu.VMEM((_CH, F), jnp.float32),
            pltpu.VMEM((_CH, F), jnp.float32),
            pltpu.VMEM((1, 16), jnp.int32),
        ],
    )
    def sc_kernel(x_hbm, cuts_hbm, fill_hbm, out_hbm, fillbuf, stage, cutbuf):
        wid = lax.axis_index("c") * 16 + lax.axis_index("s")
        b = wid // SEG
        r0 = (wid % SEG) * R
        pltpu.sync_copy(fill_hbm, fillbuf)
        pltpu.sync_copy(cuts_hbm.at[wid], cutbuf)
        cut = cutbuf[0, :][0]

        n_full = cut // _CH
        frac = cut - n_full * _CH
        has_b = (frac > 0).astype(jnp.int32)

        def copy_chunk(i, c):
            row = r0 + i * _CH
            pltpu.sync_copy(x_hbm.at[b, pl.ds(row, _CH)], stage)
            pltpu.sync_copy(stage, out_hbm.at[b, pl.ds(row, _CH)])
            return c

        lax.fori_loop(0, n_full, copy_chunk, 0)

        @pl.when(frac > 0)
        def _():
            row = r0 + n_full * _CH
            pltpu.sync_copy(x_hbm.at[b, pl.ds(row, _CH)], stage)

            def patch(r, c):
                for j in range(F // 16):
                    stage[r, pl.ds(j * 16, 16)] = fillbuf[0, pl.ds(j * 16, 16)]
                return c

            lax.fori_loop(frac, _CH, patch, 0)
            pltpu.sync_copy(stage, out_hbm.at[b, pl.ds(row, _CH)])

        def fill_chunk(i, c):
            row = r0 + i * _CH
            pltpu.sync_copy(fillbuf, out_hbm.at[b, pl.ds(row, _CH)])
            return c

        lax.fori_loop(n_full + has_b, n_chunks, fill_chunk, 0)

    return sc_kernel


def kernel(x, lens, output_mask):
    B, S, F = x.shape
    NW = 32
    SEG = NW // B
    R = S // SEG
    lens_i = lens.astype(jnp.int32)
    wids = jnp.arange(NW, dtype=jnp.int32)
    cut = jnp.clip(lens_i[wids // SEG] - (wids % SEG) * R, 0, R)
    cuts3 = jnp.broadcast_to(cut[:, None, None], (NW, 1, 16))
    fill = jnp.broadcast_to(output_mask[None, :], (_CH, F))
    return _make_sc_kernel(B, S, F, NW, SEG, R)(x, cuts3, fill)


# speedup vs baseline: 9.6238x; 1.6680x over previous
import functools

import jax
import jax.numpy as jnp
from jax import lax
from jax.experimental import pallas as pl
from jax.experimental.pallas import tpu as pltpu
from jax.experimental.pallas import tpu_sc as plsc

_CH = 32
_WPB = 8


def _make_sc_kernel(B, S, F, NW):
    mesh = plsc.VectorSubcoreMesh(core_axis_name="c", subcore_axis_name="s")
    n_local = S // _CH // _WPB
    stride = _CH * _WPB

    @functools.partial(
        pl.kernel,
        out_type=jax.ShapeDtypeStruct((B, S, F), jnp.float32),
        mesh=mesh,
        scratch_types=[
            pltpu.VMEM((_CH, F), jnp.float32),
            pltpu.VMEM((_CH, F), jnp.float32),
            pltpu.VMEM((_CH, F), jnp.float32),
            pltpu.VMEM((1, 16), jnp.int32),
            pltpu.SemaphoreType.DMA,
            pltpu.SemaphoreType.DMA,
            pltpu.SemaphoreType.DMA,
            pltpu.SemaphoreType.DMA,
            pltpu.SemaphoreType.DMA,
        ],
    )
    def sc_kernel(x_hbm, params_hbm, fill_hbm, out_hbm,
                  fillbuf, stage0, stage1, pbuf, sf, si0, si1, so0, so1):
        wid = lax.axis_index("c") * 16 + lax.axis_index("s")
        b = wid // _WPB
        seg = wid % _WPB
        pltpu.sync_copy(fill_hbm, fillbuf)
        pltpu.sync_copy(params_hbm.at[wid], pbuf)
        v = pbuf[0, :]
        n_copy = v[0]
        frac = v[1]
        owner = (frac > 0).astype(jnp.int32)

        def rowof(i):
            return seg * _CH + i * stride

        def cin(i, stg, sem):
            return pltpu.make_async_copy(
                x_hbm.at[b, pl.ds(rowof(i), _CH)], stg, sem)

        def cout(i, stg, sem):
            return pltpu.make_async_copy(
                stg, out_hbm.at[b, pl.ds(rowof(i), _CH)], sem)

        def fdma(i):
            return pltpu.make_async_copy(
                fillbuf, out_hbm.at[b, pl.ds(rowof(i), _CH)], sf)

        n_fill0 = n_copy + owner

        def fire_fill(i, c):
            fdma(i).start()
            return c

        lax.fori_loop(n_fill0, n_local, fire_fill, 0)

        @pl.when(n_copy > 0)
        def _():
            cin(0, stage0, si0).start()

        @pl.when(n_copy > 1)
        def _():
            cin(1, stage1, si1).start()

        def copy_body(i, c):
            even = i % 2 == 0

            @pl.when(even)
            def _():
                cin(i, stage0, si0).wait()
                cout(i, stage0, so0).start()

                @pl.when(i + 2 < n_copy)
                def _():
                    cout(i, stage0, so0).wait()
                    cin(i + 2, stage0, si0).start()

            @pl.when(jnp.logical_not(even))
            def _():
                cin(i, stage1, si1).wait()
                cout(i, stage1, so1).start()

                @pl.when(i + 2 < n_copy)
                def _():
                    cout(i, stage1, so1).wait()
                    cin(i + 2, stage1, si1).start()

            return c

        lax.fori_loop(0, n_copy, copy_body, 0)

        def drain(i):
            @pl.when(i % 2 == 0)
            def _():
                cout(i, stage0, so0).wait()

            @pl.when(i % 2 == 1)
            def _():
                cout(i, stage1, so1).wait()

        @pl.when(n_copy > 1)
        def _():
            drain(n_copy - 2)

        @pl.when(n_copy > 0)
        def _():
            drain(n_copy - 1)

        @pl.when(frac > 0)
        def _():
            row = rowof(n_copy)
            pltpu.sync_copy(x_hbm.at[b, pl.ds(row, _CH)], stage0)

            def patch(r, c):
                for j in range(F // 16):
                    stage0[r, pl.ds(j * 16, 16)] = fillbuf[0, pl.ds(j * 16, 16)]
                return c

            lax.fori_loop(frac, _CH, patch, 0)
            pltpu.sync_copy(stage0, out_hbm.at[b, pl.ds(row, _CH)])

        def drain_fill(i, c):
            fdma(i).wait()
            return c

        lax.fori_loop(n_fill0, n_local, drain_fill, 0)

    return sc_kernel


def kernel(x, lens, output_mask):
    B, S, F = x.shape
    NW = B * _WPB
    lens_i = lens.astype(jnp.int32)
    wids = jnp.arange(NW, dtype=jnp.int32)
    cut = jnp.clip(lens_i[wids // _WPB], 0, S)
    gc = cut // _CH
    frac_b = cut - gc * _CH
    seg = wids % _WPB
    n_copy = jnp.clip((gc - seg + (_WPB - 1)) // _WPB, 0, S // _CH // _WPB)
    owner = (frac_b > 0) & (gc % _WPB == seg)
    frac = jnp.where(owner, frac_b, 0)
    params = jnp.stack([n_copy, frac], axis=1)
    params = jnp.pad(params, ((0, 0), (0, 14)))[:, None, :]
    fill = jnp.broadcast_to(output_mask[None, :], (_CH, F))
    return _make_sc_kernel(B, S, F, NW)(x, params, fill)
